# baseline (speedup 1.0000x reference)
import functools

import jax
import jax.numpy as jnp
from jax import lax
from jax.experimental import pallas as pl
from jax.experimental.pallas import tpu as pltpu
from jax.experimental.pallas import tpu_sc as plsc

H = 256
ATOM_FDIM = 128
BOND_FDIM = 144
N_ATOMS = 10000
N_BONDS = 320000
MAX_NB = 32
N_MOLS = 500
ATOMS_PER_MOL = 20
DEPTH = 3

NC, NS = 2, 16
NW = NC * NS
NA_PAD = 10240
A_W = NA_PAD // NW
A_CHUNK = 4
A_NCHUNK = A_W // A_CHUNK
B_W = N_BONDS // NW
B_CHUNK = 80
B_NCHUNK = B_W // B_CHUNK

_mesh = functools.partial(
    plsc.VectorSubcoreMesh,
    core_axis_name="c", subcore_axis_name="s", num_cores=NC, num_subcores=NS)


def _wid():
    return lax.axis_index("s") * NC + lax.axis_index("c")


@functools.partial(
    pl.kernel,
    out_type=jax.ShapeDtypeStruct((NA_PAD, H), jnp.float32),
    mesh=_mesh(),
    scratch_types=[
        pltpu.VMEM((A_CHUNK * MAX_NB,), jnp.int32),
        pltpu.VMEM((A_CHUNK * MAX_NB, H), jnp.float32),
        pltpu.VMEM((A_CHUNK, H), jnp.float32),
        pltpu.SemaphoreType.DMA,
    ],
)
def _sc_segsum(pre_hbm, a2b_hbm, am_hbm, idx_v, rows_v, out_v, sem):
    w = _wid()

    def chunk(c, _):
        base = w * A_W + c * A_CHUNK
        pltpu.sync_copy(a2b_hbm.at[pl.ds(base * MAX_NB, A_CHUNK * MAX_NB)],
                        idx_v)
        pltpu.async_copy(pre_hbm.at[idx_v], rows_v, sem).wait()

        def one_atom(a, _):
            def row(r, acc):
                i = a * MAX_NB + r
                return tuple(
                    acc[k] + jnp.maximum(rows_v[i, pl.ds(k * 16, 16)], 0.0)
                    for k in range(H // 16))
            acc = lax.fori_loop(
                0, MAX_NB, row,
                tuple(jnp.zeros((16,), jnp.float32) for _ in range(H // 16)))
            for k in range(H // 16):
                out_v[a, pl.ds(k * 16, 16)] = acc[k]
            return 0

        lax.fori_loop(0, A_CHUNK, one_atom, 0)
        pltpu.sync_copy(out_v, am_hbm.at[pl.ds(base, A_CHUNK)])
        return 0

    lax.fori_loop(0, A_NCHUNK, chunk, 0)


@functools.partial(
    pl.kernel,
    out_type=jax.ShapeDtypeStruct((N_BONDS, H), jnp.float32),
    mesh=_mesh(),
    scratch_types=[
        pltpu.VMEM((B_CHUNK,), jnp.int32),
        pltpu.VMEM((B_CHUNK,), jnp.int32),
        pltpu.VMEM((B_CHUNK, H), jnp.float32),
        pltpu.VMEM((B_CHUNK, H), jnp.float32),
        pltpu.SemaphoreType.DMA,
        pltpu.SemaphoreType.DMA,
    ],
)
def _sc_combine(pre_hbm, am_hbm, b2a_hbm, b2revb_hbm, t_hbm,
                idxa_v, idxr_v, am_v, rev_v, sema, semr):
    w = _wid()

    def chunk(c, _):
        base = w * B_W + c * B_CHUNK
        pltpu.sync_copy(b2a_hbm.at[pl.ds(base, B_CHUNK)], idxa_v)
        pltpu.sync_copy(b2revb_hbm.at[pl.ds(base, B_CHUNK)], idxr_v)
        cpa = pltpu.async_copy(am_hbm.at[idxa_v], am_v, sema)
        cpr = pltpu.async_copy(pre_hbm.at[idxr_v], rev_v, semr)
        cpa.wait()
        cpr.wait()

        def row(r, _):
            for k in range(H // 16):
                s = pl.ds(k * 16, 16)
                rev_v[r, s] = jnp.maximum(rev_v[r, s], 0.0) - am_v[r, s]
            return 0

        lax.fori_loop(0, B_CHUNK, row, 0)
        pltpu.sync_copy(rev_v, t_hbm.at[pl.ds(base, B_CHUNK)])
        return 0

    lax.fori_loop(0, B_NCHUNK, chunk, 0)


BM = 2000


def _mm_body(x_ref, w_ref, o_ref):
    o_ref[...] = jnp.dot(x_ref[...], w_ref[...],
                         preferred_element_type=jnp.float32)


def _tc_in_proj(f_bonds, w_t):
    return pl.pallas_call(
        _mm_body,
        grid=(N_BONDS // BM,),
        in_specs=[
            pl.BlockSpec((BM, BOND_FDIM), lambda i: (i, 0)),
            pl.BlockSpec((BOND_FDIM, H), lambda i: (0, 0)),
        ],
        out_specs=pl.BlockSpec((BM, H), lambda i: (i, 0)),
        out_shape=jax.ShapeDtypeStruct((N_BONDS, H), jnp.float32),
    )(f_bonds, w_t)


def _addmm_body(x_ref, w_ref, b_ref, o_ref):
    o_ref[...] = b_ref[...] + jnp.dot(x_ref[...], w_ref[...],
                                      preferred_element_type=jnp.float32)


def _tc_update(t, whn_t, inp):
    return pl.pallas_call(
        _addmm_body,
        grid=(N_BONDS // BM,),
        in_specs=[
            pl.BlockSpec((BM, H), lambda i: (i, 0)),
            pl.BlockSpec((H, H), lambda i: (0, 0)),
            pl.BlockSpec((BM, H), lambda i: (i, 0)),
        ],
        out_specs=pl.BlockSpec((BM, H), lambda i: (i, 0)),
        out_shape=jax.ShapeDtypeStruct((N_BONDS, H), jnp.float32),
    )(t, whn_t, inp)


def _readout_body(fa_ref, am_ref, wo1_ref, wo2_ref, bo_ref, r0_ref, o_ref):
    hid = jnp.dot(fa_ref[...], wo1_ref[...], preferred_element_type=jnp.float32)
    hid += jnp.dot(am_ref[...], wo2_ref[...], preferred_element_type=jnp.float32)
    hid = jnp.maximum(hid + bo_ref[...], 0.0)
    o_ref[...] = jnp.dot(r0_ref[...], hid, preferred_element_type=jnp.float32)


def _tc_readout(f_atoms, am, wo1_t, wo2_t, b_o2d, r0):
    return pl.pallas_call(
        _readout_body,
        out_shape=jax.ShapeDtypeStruct((N_MOLS, H), jnp.float32),
    )(f_atoms, am, wo1_t, wo2_t, b_o2d, r0)


def kernel(f_atoms, f_bonds, a2b, b2a, b2revb, W_i, W_h, W_o, b_o):
    a2b_flat = jnp.zeros((NA_PAD, MAX_NB), jnp.int32).at[:N_ATOMS].set(
        a2b).reshape(-1)
    wi_t = W_i.T
    whn_t = -W_h.T
    wo1_t = W_o[:, :ATOM_FDIM].T
    wo2_t = W_o[:, ATOM_FDIM:].T
    b_o2d = b_o.reshape(1, H)
    r0 = jnp.kron(jnp.eye(N_MOLS, dtype=jnp.float32),
                  jnp.full((1, ATOMS_PER_MOL), 1.0 / ATOMS_PER_MOL,
                           jnp.float32))

    pre = _tc_in_proj(f_bonds, wi_t)
    inp = pre
    for _ in range(DEPTH - 1):
        am = _sc_segsum(pre, a2b_flat)
        t = _sc_combine(pre, am, b2a, b2revb)
        pre = _tc_update(t, whn_t, inp)
    am = _sc_segsum(pre, a2b_flat)
    return _tc_readout(f_atoms, am[:N_ATOMS], wo1_t, wo2_t, b_o2d, r0)


# speedup vs baseline: 1.3058x; 1.3058x over previous
import functools

import jax
import jax.numpy as jnp
from jax import lax
from jax.experimental import pallas as pl
from jax.experimental.pallas import tpu as pltpu
from jax.experimental.pallas import tpu_sc as plsc

H = 256
ATOM_FDIM = 128
BOND_FDIM = 144
N_ATOMS = 10000
N_BONDS = 320000
MAX_NB = 32
N_MOLS = 500
ATOMS_PER_MOL = 20
DEPTH = 3

NC, NS = 2, 16
NW = NC * NS
NA_PAD = 10240
A_W = NA_PAD // NW
A_CHUNK = 4
A_NCHUNK = A_W // A_CHUNK
B_W = N_BONDS // NW
B_CHUNK = 40
B_NCHUNK = B_W // B_CHUNK

_mesh = functools.partial(
    plsc.VectorSubcoreMesh,
    core_axis_name="c", subcore_axis_name="s", num_cores=NC, num_subcores=NS)


def _wid():
    return lax.axis_index("s") * NC + lax.axis_index("c")


@functools.partial(
    pl.kernel,
    out_type=jax.ShapeDtypeStruct((NA_PAD, H), jnp.float32),
    mesh=_mesh(),
    scratch_types=[
        pltpu.VMEM((A_W * MAX_NB,), jnp.int32),
        [pltpu.VMEM((A_CHUNK * MAX_NB, H), jnp.float32) for _ in range(2)],
        [pltpu.VMEM((A_CHUNK, H), jnp.float32) for _ in range(2)],
        [pltpu.SemaphoreType.DMA for _ in range(2)],
        [pltpu.SemaphoreType.DMA for _ in range(2)],
    ],
)
def _sc_segsum(pre_hbm, a2b_hbm, am_hbm, idx_v, rows_v, out_v, gsem, wsem):
    w = _wid()
    nidx = A_CHUNK * MAX_NB

    def gather(c, s):
        pltpu.async_copy(
            pre_hbm.at[idx_v.at[pl.ds(c * nidx, nidx)]], rows_v[s], gsem[s])

    pltpu.sync_copy(a2b_hbm.at[pl.ds(w * A_W * MAX_NB, A_W * MAX_NB)], idx_v)
    gather(0, 0)
    gather(1, 1)

    def chunk(c, s):
        pltpu.make_async_copy(
            pre_hbm.at[idx_v.at[pl.ds(c * nidx, nidx)]],
            rows_v[s], gsem[s]).wait()
        base = w * A_W + c * A_CHUNK

        @pl.when(c >= 2)
        def _():
            pltpu.make_async_copy(
                out_v[s],
                am_hbm.at[pl.ds(base - 2 * A_CHUNK, A_CHUNK)], wsem[s]).wait()
        for a in range(A_CHUNK):
            def row(r, acc):
                return tuple(
                    acc[k] + jnp.maximum(
                        rows_v[s][a * MAX_NB + r, pl.ds(k * 16, 16)], 0.0)
                    for k in range(H // 16))
            acc = lax.fori_loop(
                0, MAX_NB, row,
                tuple(jnp.zeros((16,), jnp.float32) for _ in range(H // 16)))
            for k in range(H // 16):
                out_v[s][a, pl.ds(k * 16, 16)] = acc[k]

        @pl.when(c + 2 < A_NCHUNK)
        def _():
            gather(c + 2, s)
        pltpu.async_copy(out_v[s], am_hbm.at[pl.ds(base, A_CHUNK)], wsem[s])

    def pair(g, _):
        chunk(2 * g, 0)
        chunk(2 * g + 1, 1)
        return 0

    lax.fori_loop(0, A_NCHUNK // 2, pair, 0)
    for s, c in ((0, A_NCHUNK - 2), (1, A_NCHUNK - 1)):
        base = w * A_W + c * A_CHUNK
        pltpu.make_async_copy(
            out_v[s], am_hbm.at[pl.ds(base, A_CHUNK)], wsem[s]).wait()


@functools.partial(
    pl.kernel,
    out_type=jax.ShapeDtypeStruct((N_BONDS, H), jnp.float32),
    mesh=_mesh(),
    scratch_types=[
        pltpu.VMEM((B_W,), jnp.int32),
        pltpu.VMEM((B_W,), jnp.int32),
        [pltpu.VMEM((B_CHUNK, H), jnp.float32) for _ in range(2)],
        [pltpu.VMEM((B_CHUNK, H), jnp.float32) for _ in range(2)],
        [pltpu.VMEM((B_CHUNK, H), jnp.float32) for _ in range(2)],
        [pltpu.SemaphoreType.DMA for _ in range(2)],
        [pltpu.SemaphoreType.DMA for _ in range(2)],
        [pltpu.SemaphoreType.DMA for _ in range(2)],
    ],
)
def _sc_combine(pre_hbm, am_hbm, b2a_hbm, b2revb_hbm, t_hbm,
                idxa_v, idxr_v, am_v, rev_v, out_v, sema, semr, semw):
    w = _wid()

    def gathers(c, s):
        pltpu.async_copy(
            am_hbm.at[idxa_v.at[pl.ds(c * B_CHUNK, B_CHUNK)]], am_v[s],
            sema[s])
        pltpu.async_copy(
            pre_hbm.at[idxr_v.at[pl.ds(c * B_CHUNK, B_CHUNK)]], rev_v[s],
            semr[s])

    pltpu.sync_copy(b2a_hbm.at[pl.ds(w * B_W, B_W)], idxa_v)
    pltpu.sync_copy(b2revb_hbm.at[pl.ds(w * B_W, B_W)], idxr_v)
    gathers(0, 0)
    gathers(1, 1)

    def chunk(c, s):
        pltpu.make_async_copy(
            am_hbm.at[idxa_v.at[pl.ds(c * B_CHUNK, B_CHUNK)]], am_v[s],
            sema[s]).wait()
        pltpu.make_async_copy(
            pre_hbm.at[idxr_v.at[pl.ds(c * B_CHUNK, B_CHUNK)]], rev_v[s],
            semr[s]).wait()
        base = w * B_W + c * B_CHUNK

        @pl.when(c >= 2)
        def _():
            pltpu.make_async_copy(
                out_v[s],
                t_hbm.at[pl.ds(base - 2 * B_CHUNK, B_CHUNK)], semw[s]).wait()

        def row(r, _):
            for k in range(H // 16):
                sl = pl.ds(k * 16, 16)
                out_v[s][r, sl] = (jnp.maximum(rev_v[s][r, sl], 0.0)
                                   - am_v[s][r, sl])
            return 0

        lax.fori_loop(0, B_CHUNK, row, 0)

        @pl.when(c + 2 < B_NCHUNK)
        def _():
            gathers(c + 2, s)
        pltpu.async_copy(out_v[s], t_hbm.at[pl.ds(base, B_CHUNK)], semw[s])

    def pair(g, _):
        chunk(2 * g, 0)
        chunk(2 * g + 1, 1)
        return 0

    lax.fori_loop(0, B_NCHUNK // 2, pair, 0)
    for s, c in ((0, B_NCHUNK - 2), (1, B_NCHUNK - 1)):
        base = w * B_W + c * B_CHUNK
        pltpu.make_async_copy(
            out_v[s], t_hbm.at[pl.ds(base, B_CHUNK)], semw[s]).wait()


BM = 2000


def _mm_body(x_ref, w_ref, o_ref):
    o_ref[...] = jnp.dot(x_ref[...], w_ref[...],
                         preferred_element_type=jnp.float32)


def _tc_in_proj(f_bonds, w_t):
    return pl.pallas_call(
        _mm_body,
        grid=(N_BONDS // BM,),
        in_specs=[
            pl.BlockSpec((BM, BOND_FDIM), lambda i: (i, 0)),
            pl.BlockSpec((BOND_FDIM, H), lambda i: (0, 0)),
        ],
        out_specs=pl.BlockSpec((BM, H), lambda i: (i, 0)),
        out_shape=jax.ShapeDtypeStruct((N_BONDS, H), jnp.float32),
    )(f_bonds, w_t)


def _addmm_body(x_ref, w_ref, b_ref, o_ref):
    o_ref[...] = b_ref[...] + jnp.dot(x_ref[...], w_ref[...],
                                      preferred_element_type=jnp.float32)


def _tc_update(t, whn_t, inp):
    return pl.pallas_call(
        _addmm_body,
        grid=(N_BONDS // BM,),
        in_specs=[
            pl.BlockSpec((BM, H), lambda i: (i, 0)),
            pl.BlockSpec((H, H), lambda i: (0, 0)),
            pl.BlockSpec((BM, H), lambda i: (i, 0)),
        ],
        out_specs=pl.BlockSpec((BM, H), lambda i: (i, 0)),
        out_shape=jax.ShapeDtypeStruct((N_BONDS, H), jnp.float32),
    )(t, whn_t, inp)


def _readout_body(fa_ref, am_ref, wo1_ref, wo2_ref, bo_ref, r0_ref, o_ref):
    hid = jnp.dot(fa_ref[...], wo1_ref[...], preferred_element_type=jnp.float32)
    hid += jnp.dot(am_ref[...], wo2_ref[...], preferred_element_type=jnp.float32)
    hid = jnp.maximum(hid + bo_ref[...], 0.0)
    o_ref[...] = jnp.dot(r0_ref[...], hid, preferred_element_type=jnp.float32)


def _tc_readout(f_atoms, am, wo1_t, wo2_t, b_o2d, r0):
    return pl.pallas_call(
        _readout_body,
        out_shape=jax.ShapeDtypeStruct((N_MOLS, H), jnp.float32),
    )(f_atoms, am, wo1_t, wo2_t, b_o2d, r0)


def kernel(f_atoms, f_bonds, a2b, b2a, b2revb, W_i, W_h, W_o, b_o):
    a2b_flat = jnp.zeros((NA_PAD, MAX_NB), jnp.int32).at[:N_ATOMS].set(
        a2b).reshape(-1)
    wi_t = W_i.T
    whn_t = -W_h.T
    wo1_t = W_o[:, :ATOM_FDIM].T
    wo2_t = W_o[:, ATOM_FDIM:].T
    b_o2d = b_o.reshape(1, H)
    r0 = jnp.kron(jnp.eye(N_MOLS, dtype=jnp.float32),
                  jnp.full((1, ATOMS_PER_MOL), 1.0 / ATOMS_PER_MOL,
                           jnp.float32))

    pre = _tc_in_proj(f_bonds, wi_t)
    inp = pre
    for _ in range(DEPTH - 1):
        am = _sc_segsum(pre, a2b_flat)
        t = _sc_combine(pre, am, b2a, b2revb)
        pre = _tc_update(t, whn_t, inp)
    am = _sc_segsum(pre, a2b_flat)
    return _tc_readout(f_atoms, am[:N_ATOMS], wo1_t, wo2_t, b_o2d, r0)
